# baseline (speedup 1.0000x reference)
import functools

import jax
import jax.numpy as jnp
from jax import lax
from jax.experimental import pallas as pl
from jax.experimental.pallas import tpu as pltpu
from jax.experimental.pallas import tpu_sc as plsc

_NC = 2
_NS = 16
_NW = _NC * _NS
_CHUNK = 128
_D = 16
_TSUB = 8
_TLANE = 128


def _sc_gather_t(table_flat, idx3, ntile):
    NW, K, C = idx3.shape
    D = _D
    cols_per_w = K * C
    B = (NW // 2) * cols_per_w
    npt = _TSUB * _TLANE
    mesh = plsc.VectorSubcoreMesh(core_axis_name="c", subcore_axis_name="s")

    @functools.partial(
        pl.kernel,
        mesh=mesh,
        out_type=jax.ShapeDtypeStruct((2 * D, B), jnp.float32),
        scratch_types=[
            pltpu.VMEM((K, C), jnp.int32),
            pltpu.VMEM((D, K, C), jnp.int32),
            pltpu.VMEM((D, cols_per_w), jnp.float32),
            pltpu.SemaphoreType.DMA,
        ],
    )
    def gather_k(tab_hbm, idx_hbm, out_hbm, idx_v, fid_v, rows_v, gsem):
        wid = lax.axis_index("s") * _NC + lax.axis_index("c")
        row0 = (wid // 16) * D
        col0 = (wid % 16) * cols_per_w
        pltpu.sync_copy(idx_hbm.at[wid], idx_v)
        cps = []
        for j in range(K):
            for t in range(C // 16):
                iv = idx_v[j, t * 16:(t + 1) * 16]
                base = ((iv >> 7) << 10) + (iv & (_TLANE - 1))
                for d in range(D):
                    off = (d // _TSUB) * (ntile * npt) + (d % _TSUB) * _TLANE
                    fid_v[d, j, t * 16:(t + 1) * 16] = base + off
            for d in range(D):
                cps.append(pltpu.async_copy(
                    tab_hbm.at[fid_v.at[d, j]],
                    rows_v.at[d, pl.ds(j * C, C)], gsem))
        for cp in cps:
            cp.wait()
        pltpu.sync_copy(rows_v,
                        out_hbm.at[pl.ds(row0, D), pl.ds(col0, cols_per_w)])

    return gather_k(table_flat, idx3)


def _mlp_t_body(hT_ref, W1_ref, b1_ref, g1_ref, be1_ref, W2_ref, b2_ref,
                g2_ref, be2_ref, Wfc_ref, bfc_ref, out_ref):
    hT = hT_ref[:]
    H1 = lax.dot_general(W1_ref[:], hT, (((0,), (0,)), ((), ())),
                         preferred_element_type=jnp.float32)
    H1 = H1 + b1_ref[:][:, None]
    m1 = jnp.mean(H1, axis=1, keepdims=True)
    v1 = jnp.mean((H1 - m1) ** 2, axis=1, keepdims=True)
    X1 = g1_ref[:][:, None] * (H1 - m1) * lax.rsqrt(v1 + 1e-5)
    X1 = jnp.maximum(X1 + be1_ref[:][:, None], 0.0)
    H2 = lax.dot_general(W2_ref[:], X1, (((0,), (0,)), ((), ())),
                         preferred_element_type=jnp.float32)
    H2 = H2 + b2_ref[:][:, None]
    m2 = jnp.mean(H2, axis=1, keepdims=True)
    v2 = jnp.mean((H2 - m2) ** 2, axis=1, keepdims=True)
    X2 = g2_ref[:][:, None] * (H2 - m2) * lax.rsqrt(v2 + 1e-5)
    X2 = jnp.maximum(X2 + be2_ref[:][:, None], 0.0)
    gmf = hT[0:_D, :] * hT[_D:2 * _D, :]
    w = Wfc_ref[:]
    acc = lax.dot_general(w[:_D, :], gmf, (((0,), (0,)), ((), ())),
                          preferred_element_type=jnp.float32)
    acc = acc + lax.dot_general(w[_D:, :], X2, (((0,), (0,)), ((), ())),
                                preferred_element_type=jnp.float32)
    out_ref[:] = acc + bfc_ref[:][:, None]


def _tc_mlp_t(hT, W1, b1, g1, be1, W2, b2, g2, be2, Wfc, bfc):
    B = hT.shape[1]
    return pl.pallas_call(
        _mlp_t_body,
        out_shape=jax.ShapeDtypeStruct((1, B), jnp.float32),
    )(hT, W1, b1, g1, be1, W2, b2, g2, be2, Wfc, bfc)


def kernel(x, emb_table, W1, b1, g1, be1, W2, b2, g2, be2, Wfc, bfc):
    B = x.shape[0]
    V = emb_table.shape[0]
    ntile = V // _TLANE
    offsets = jnp.array([0, V // 2], dtype=x.dtype)
    idx = x + offsets[None, :]
    idx_col = jnp.concatenate([idx[:, 0], idx[:, 1]])
    idx3 = idx_col.reshape(_NW, (2 * B) // (_NW * _CHUNK), _CHUNK)
    table_flat = (emb_table.T.reshape(_D // _TSUB, _TSUB, ntile, _TLANE)
                  .transpose(0, 2, 1, 3).reshape(-1))
    hT = _sc_gather_t(table_flat, idx3, ntile)
    out = _tc_mlp_t(hT, W1, b1, g1, be1, W2, b2, g2, be2, Wfc, bfc)
    return out.reshape(B)


# speedup vs baseline: 1.0519x; 1.0519x over previous
import functools

import jax
import jax.numpy as jnp
from jax import lax
from jax.experimental import pallas as pl
from jax.experimental.pallas import tpu as pltpu
from jax.experimental.pallas import tpu_sc as plsc

_NC = 2
_NS = 16
_NW = _NC * _NS
_CHUNK = 128
_D = 16
_TSUB = 8
_TLANE = 128


def _sc_gather_t(table_flat, fidx):
    NW, D, K, C = fidx.shape
    cols_per_w = K * C
    B = (NW // 2) * cols_per_w
    mesh = plsc.VectorSubcoreMesh(core_axis_name="c", subcore_axis_name="s")

    @functools.partial(
        pl.kernel,
        mesh=mesh,
        out_type=jax.ShapeDtypeStruct((2 * D, B), jnp.float32),
        scratch_types=[
            pltpu.VMEM((D, K, C), jnp.int32),
            pltpu.VMEM((D, cols_per_w), jnp.float32),
            pltpu.SemaphoreType.DMA,
        ],
    )
    def gather_k(tab_hbm, fidx_hbm, out_hbm, idx_v, rows_v, gsem):
        wid = lax.axis_index("s") * _NC + lax.axis_index("c")
        row0 = (wid // 16) * D
        col0 = (wid % 16) * cols_per_w
        pltpu.sync_copy(fidx_hbm.at[wid], idx_v)
        cps = []
        for d in range(D):
            for j in range(K):
                cps.append(pltpu.async_copy(
                    tab_hbm.at[idx_v.at[d, j]],
                    rows_v.at[d, pl.ds(j * C, C)], gsem))
        for cp in cps:
            cp.wait()
        pltpu.sync_copy(rows_v,
                        out_hbm.at[pl.ds(row0, D), pl.ds(col0, cols_per_w)])

    return gather_k(table_flat, fidx)


def _mlp_t_body(hT_ref, W1_ref, b1_ref, g1_ref, be1_ref, W2_ref, b2_ref,
                g2_ref, be2_ref, Wfc_ref, bfc_ref, out_ref):
    hT = hT_ref[:]
    H1 = lax.dot_general(W1_ref[:], hT, (((0,), (0,)), ((), ())),
                         preferred_element_type=jnp.float32)
    H1 = H1 + b1_ref[:][:, None]
    m1 = jnp.mean(H1, axis=1, keepdims=True)
    q1 = jnp.mean(H1 * H1, axis=1, keepdims=True)
    s1 = g1_ref[:][:, None] * lax.rsqrt(q1 - m1 * m1 + 1e-5)
    X1 = jnp.maximum(H1 * s1 + (be1_ref[:][:, None] - s1 * m1), 0.0)
    H2 = lax.dot_general(W2_ref[:], X1, (((0,), (0,)), ((), ())),
                         preferred_element_type=jnp.float32)
    H2 = H2 + b2_ref[:][:, None]
    m2 = jnp.mean(H2, axis=1, keepdims=True)
    q2 = jnp.mean(H2 * H2, axis=1, keepdims=True)
    s2 = g2_ref[:][:, None] * lax.rsqrt(q2 - m2 * m2 + 1e-5)
    X2 = jnp.maximum(H2 * s2 + (be2_ref[:][:, None] - s2 * m2), 0.0)
    gmf = hT[0:_D, :] * hT[_D:2 * _D, :]
    w = Wfc_ref[:]
    acc = lax.dot_general(w[:_D, :], gmf, (((0,), (0,)), ((), ())),
                          preferred_element_type=jnp.float32)
    acc = acc + lax.dot_general(w[_D:, :], X2, (((0,), (0,)), ((), ())),
                                preferred_element_type=jnp.float32)
    out_ref[:] = acc + bfc_ref[:][:, None]


def _tc_mlp_t(hT, W1, b1, g1, be1, W2, b2, g2, be2, Wfc, bfc):
    B = hT.shape[1]
    return pl.pallas_call(
        _mlp_t_body,
        out_shape=jax.ShapeDtypeStruct((1, B), jnp.float32),
    )(hT, W1, b1, g1, be1, W2, b2, g2, be2, Wfc, bfc)


def kernel(x, emb_table, W1, b1, g1, be1, W2, b2, g2, be2, Wfc, bfc):
    B = x.shape[0]
    V = emb_table.shape[0]
    ntile = V // _TLANE
    offsets = jnp.array([0, V // 2], dtype=x.dtype)
    idx = x + offsets[None, :]
    idx_col = jnp.concatenate([idx[:, 0], idx[:, 1]])
    ir = idx_col.reshape(_NW, 1, (2 * B) // (_NW * _CHUNK), _CHUNK)
    d = jnp.arange(_D, dtype=jnp.int32).reshape(1, _D, 1, 1)
    fidx = (((d // _TSUB) * ntile + (ir >> 7)) * (_TSUB * _TLANE)
            + (d % _TSUB) * _TLANE + (ir & (_TLANE - 1)))
    table_flat = (emb_table.T.reshape(_D // _TSUB, _TSUB, ntile, _TLANE)
                  .transpose(0, 2, 1, 3).reshape(-1))
    hT = _sc_gather_t(table_flat, fidx)
    out = _tc_mlp_t(hT, W1, b1, g1, be1, W2, b2, g2, be2, Wfc, bfc)
    return out.reshape(B)
